# baseline (speedup 1.0000x reference)
import jax
import jax.numpy as jnp
from jax.experimental import pallas as pl
from jax.experimental.pallas import tpu as pltpu

B = 4096
IN_DIM = 128
OUT_DIM = 256
N_BINS = 50
TAU = 10.0

CH = 32
NBUF = 16
NCHUNKS = B // CH


def _encoder(x_ref, wt_ref, b_ref, lat_hbm, spk_hbm,
             spk_buf, lat_buf, spk_sem, lat_sem):
    bins = jax.lax.broadcasted_iota(jnp.int32, (CH, N_BINS, OUT_DIM), 1)

    def spk_copy(i, slot):
        return pltpu.make_async_copy(
            spk_buf.at[slot], spk_hbm.at[pl.ds(i * CH, CH)], spk_sem.at[slot])

    def lat_copy(i, slot):
        return pltpu.make_async_copy(
            lat_buf.at[slot], lat_hbm.at[pl.ds(i * CH, CH)], lat_sem.at[slot])

    def body(i, carry):
        slot = jax.lax.rem(i, NBUF)

        @pl.when(i >= NBUF)
        def _():
            spk_copy(i - NBUF, slot).wait()
            lat_copy(i - NBUF, slot).wait()

        xs = x_ref[pl.ds(i * CH, CH), :]
        rates = jax.lax.dot_general(
            xs, wt_ref[...],
            dimension_numbers=(((1,), (0,)), ((), ())),
            preferred_element_type=jnp.float32,
        ) + b_ref[...]
        lat = jnp.clip(N_BINS * jnp.exp(-rates / TAU), 1, N_BINS - 1
                       ).astype(jnp.int32)
        lat_buf[slot] = lat
        spk_buf[slot] = (bins == lat[:, None, :]).astype(jnp.float32)

        spk_copy(i, slot).start()
        lat_copy(i, slot).start()
        return carry

    jax.lax.fori_loop(0, NCHUNKS, body, 0)

    def drain(j, carry):
        i = NCHUNKS - NBUF + j
        slot = jax.lax.rem(i, NBUF)
        spk_copy(i, slot).wait()
        lat_copy(i, slot).wait()
        return carry

    jax.lax.fori_loop(0, NBUF, drain, 0)


def kernel(x, W, b):
    wt = W.T
    b2 = b.reshape(1, OUT_DIM)
    lat, spikes = pl.pallas_call(
        _encoder,
        in_specs=[
            pl.BlockSpec(memory_space=pltpu.MemorySpace.VMEM),
            pl.BlockSpec(memory_space=pltpu.MemorySpace.VMEM),
            pl.BlockSpec(memory_space=pltpu.MemorySpace.VMEM),
        ],
        out_specs=[
            pl.BlockSpec(memory_space=pltpu.MemorySpace.HBM),
            pl.BlockSpec(memory_space=pltpu.MemorySpace.HBM),
        ],
        out_shape=[
            jax.ShapeDtypeStruct((B, OUT_DIM), jnp.int32),
            jax.ShapeDtypeStruct((B, N_BINS, OUT_DIM), jnp.float32),
        ],
        scratch_shapes=[
            pltpu.VMEM((NBUF, CH, N_BINS, OUT_DIM), jnp.float32),
            pltpu.VMEM((NBUF, CH, OUT_DIM), jnp.int32),
            pltpu.SemaphoreType.DMA((NBUF,)),
            pltpu.SemaphoreType.DMA((NBUF,)),
        ],
    )(x, wt, b2)
    return (lat, spikes)


# speedup vs baseline: 1.0005x; 1.0005x over previous
import jax
import jax.numpy as jnp
from jax.experimental import pallas as pl
from jax.experimental.pallas import tpu as pltpu

B = 4096
IN_DIM = 128
OUT_DIM = 256
N_BINS = 50
TAU = 10.0

CH = 64
NBUF = 8
NCHUNKS = B // CH


def _encoder(x_ref, wt_ref, b_ref, lat_hbm, spk_hbm,
             spk_buf, lat_buf, spk_sem, lat_sem):
    bins = jax.lax.broadcasted_iota(jnp.int32, (CH, N_BINS, OUT_DIM), 1)

    def spk_copy(i, slot):
        return pltpu.make_async_copy(
            spk_buf.at[slot], spk_hbm.at[pl.ds(i * CH, CH)], spk_sem.at[slot])

    def lat_copy(i, slot):
        return pltpu.make_async_copy(
            lat_buf.at[slot], lat_hbm.at[pl.ds(i * CH, CH)], lat_sem.at[slot])

    def body(i, carry):
        slot = jax.lax.rem(i, NBUF)

        @pl.when(i >= NBUF)
        def _():
            spk_copy(i - NBUF, slot).wait()
            lat_copy(i - NBUF, slot).wait()

        xs = x_ref[pl.ds(i * CH, CH), :]
        rates = jax.lax.dot_general(
            xs, wt_ref[...],
            dimension_numbers=(((1,), (0,)), ((), ())),
            preferred_element_type=jnp.float32,
        ) + b_ref[...]
        lat = jnp.clip(N_BINS * jnp.exp(-rates / TAU), 1, N_BINS - 1
                       ).astype(jnp.int32)
        lat_buf[slot] = lat
        spk_buf[slot] = (bins == lat[:, None, :]).astype(jnp.float32)

        spk_copy(i, slot).start()
        lat_copy(i, slot).start()
        return carry

    jax.lax.fori_loop(0, NCHUNKS, body, 0)

    def drain(j, carry):
        i = NCHUNKS - NBUF + j
        slot = jax.lax.rem(i, NBUF)
        spk_copy(i, slot).wait()
        lat_copy(i, slot).wait()
        return carry

    jax.lax.fori_loop(0, NBUF, drain, 0)


def kernel(x, W, b):
    wt = W.T
    b2 = b.reshape(1, OUT_DIM)
    lat, spikes = pl.pallas_call(
        _encoder,
        in_specs=[
            pl.BlockSpec(memory_space=pltpu.MemorySpace.VMEM),
            pl.BlockSpec(memory_space=pltpu.MemorySpace.VMEM),
            pl.BlockSpec(memory_space=pltpu.MemorySpace.VMEM),
        ],
        out_specs=[
            pl.BlockSpec(memory_space=pltpu.MemorySpace.HBM),
            pl.BlockSpec(memory_space=pltpu.MemorySpace.HBM),
        ],
        out_shape=[
            jax.ShapeDtypeStruct((B, OUT_DIM), jnp.int32),
            jax.ShapeDtypeStruct((B, N_BINS, OUT_DIM), jnp.float32),
        ],
        scratch_shapes=[
            pltpu.VMEM((NBUF, CH, N_BINS, OUT_DIM), jnp.float32),
            pltpu.VMEM((NBUF, CH, OUT_DIM), jnp.int32),
            pltpu.SemaphoreType.DMA((NBUF,)),
            pltpu.SemaphoreType.DMA((NBUF,)),
        ],
    )(x, wt, b2)
    return (lat, spikes)
